# baseline (speedup 1.0000x reference)
import functools

import jax
import jax.numpy as jnp
from jax import lax
from jax.experimental import pallas as pl
from jax.experimental.pallas import tpu as pltpu
from jax.experimental.pallas import tpu_sc as plsc

_NC = 2
_NS = 16
_NW = _NC * _NS


def _build(B, F, V, D):
    N = B * F
    CHUNK = 128
    units_w = N // CHUNK // _NW
    MEGA = 8
    n_mega = units_w // MEGA
    assert N % (CHUNK * _NW) == 0 and units_w % MEGA == 0
    assert B % CHUNK == 0
    cpf = B // CHUNK

    mesh = plsc.VectorSubcoreMesh(core_axis_name="c", subcore_axis_name="s")

    @functools.partial(
        pl.kernel,
        out_type=jax.ShapeDtypeStruct((N, D), jnp.float32),
        mesh=mesh,
        compiler_params=pltpu.CompilerParams(use_tc_tiling_on_sc=False),
        scratch_types=[
            pltpu.VMEM((MEGA * CHUNK,), jnp.int32),
            pltpu.VMEM((MEGA * CHUNK, D), jnp.float32),
            pltpu.SemaphoreType.DMA,
        ],
    )
    def k(idx_hbm, tab_hbm, out_hbm, idx_v, rows_v, sem):
        wid = lax.axis_index("s") * _NC + lax.axis_index("c")

        for m in range(n_mega):
            base_u = wid * units_w + m * MEGA
            pltpu.sync_copy(
                idx_hbm.at[pl.ds(base_u * CHUNK, MEGA * CHUNK)], idx_v
            )
            copies = []
            for j in range(MEGA):
                f = (base_u + j) // cpf
                copies.append(
                    pltpu.async_copy(
                        tab_hbm.at[f].at[idx_v.at[pl.ds(j * CHUNK, CHUNK)]],
                        rows_v.at[pl.ds(j * CHUNK, CHUNK)],
                        sem,
                    )
                )
            for c in copies:
                c.wait()
            pltpu.sync_copy(
                rows_v, out_hbm.at[pl.ds(base_u * CHUNK, MEGA * CHUNK)]
            )

    return k


def kernel(indices, tables):
    B, F = indices.shape
    _, V, D = tables.shape
    idx_fm = indices.T.reshape(F * B)
    out = _build(B, F, V, D)(idx_fm, tables)
    return jnp.swapaxes(out.reshape(F, B, D), 0, 1).reshape(B, F * D)
